# baseline (speedup 1.0000x reference)
import functools

import jax
import jax.numpy as jnp
from jax import lax
from jax.experimental import pallas as pl
from jax.experimental.pallas import tpu as pltpu
from jax.experimental.pallas import tpu_sc as plsc

STATIC_VARS = 26
DIM = 64
BATCH = 4096
B = BATCH * STATIC_VARS
NC, NS = 2, 16
NW = NC * NS
BPW = B // NW
FPW = BPW // STATIC_VARS
L = 16
SLABR = 416
FPS = SLABR // STATIC_VARS
OUTW = STATIC_VARS * DIM
NSL = BPW // SLABR
K = 2
UNROLL = 2

_MESH = plsc.VectorSubcoreMesh(
    core_axis_name="c", subcore_axis_name="s", num_cores=NC, num_subcores=NS
)


@functools.partial(
    pl.kernel,
    out_type=jax.ShapeDtypeStruct((BATCH, OUTW), jnp.float32),
    mesh=_MESH,
    scratch_types=[
        pltpu.VMEM((STATIC_VARS, DIM), jnp.float32),
        pltpu.VMEM((FPW, STATIC_VARS), jnp.int32),
        pltpu.VMEM((K * SLABR * DIM,), jnp.float32),
        pltpu.SemaphoreType.DMA((K,)),
    ],
    compiler_params=pltpu.CompilerParams(
        use_tc_tiling_on_sc=True, needs_layout_passes=False
    ),
)
def _emb_lookup(table_hbm, idx_hbm, out_hbm, table_v, idx_v, bufs, osems):
    wid = lax.axis_index("s") * NC + lax.axis_index("c")
    frow0 = wid * FPW
    pltpu.sync_copy(table_hbm, table_v)
    pltpu.sync_copy(idx_hbm.at[pl.ds(frow0, FPW)], idx_v)

    lane_iota = lax.iota(jnp.int32, L)

    def drain_out(slot):
        for fr in range(FPS):
            pltpu.make_async_copy(
                bufs.at[pl.ds(slot * SLABR * DIM + fr * OUTW, OUTW)],
                out_hbm.at[frow0],
                osems.at[slot],
            ).wait()

    def body(s, _):
        slot = s % K

        @pl.when(s >= K)
        def _():
            drain_out(slot)

        sbuf = slot * SLABR * DIM

        @plsc.parallel_loop(0, FPS, unroll=UNROLL)
        def frow(fl):
            fr = s * FPS + fl
            frsplat = jnp.full((L,), fr, jnp.int32)
            for v in range(STATIC_VARS):
                rsplat = plsc.load_gather(
                    idx_v, [frsplat, jnp.full((L,), v, jnp.int32)]
                )
                obase = sbuf + (fl * STATIC_VARS + v) * DIM
                for q in range(DIM // L):
                    vals = plsc.load_gather(
                        table_v, [rsplat, lane_iota + q * L]
                    )
                    bufs[pl.ds(obase + q * L, L)] = vals

        for fr in range(FPS):
            pltpu.async_copy(
                bufs.at[pl.ds(sbuf + fr * OUTW, OUTW)],
                out_hbm.at[frow0 + s * FPS + fr],
                osems.at[slot],
            )
        return 0

    lax.fori_loop(0, NSL, body, 0)
    for t in range(K):
        drain_out((NSL - K + t) % K)


def kernel(static_input, table):
    return _emb_lookup(table.astype(jnp.float32), static_input.astype(jnp.int32))


# speedup vs baseline: 1.3950x; 1.3950x over previous
import functools

import jax
import jax.numpy as jnp
from jax import lax
from jax.experimental import pallas as pl
from jax.experimental.pallas import tpu as pltpu
from jax.experimental.pallas import tpu_sc as plsc

STATIC_VARS = 26
DIM = 64
BATCH = 4096
B = BATCH * STATIC_VARS
NC, NS = 2, 16
NW = NC * NS
BPW = B // NW
L = 16
SLABR = 416
NSL = BPW // SLABR
K = 2
UNROLL = 16

_MESH = plsc.VectorSubcoreMesh(
    core_axis_name="c", subcore_axis_name="s", num_cores=NC, num_subcores=NS
)


@functools.partial(
    pl.kernel,
    out_type=jax.ShapeDtypeStruct((BATCH, STATIC_VARS * DIM), jnp.float32),
    mesh=_MESH,
    scratch_types=[
        pltpu.VMEM((STATIC_VARS * DIM,), jnp.float32),
        pltpu.VMEM((BPW,), jnp.int32),
        pltpu.VMEM((K * SLABR * DIM,), jnp.float32),
        pltpu.SemaphoreType.DMA((K,)),
    ],
    compiler_params=pltpu.CompilerParams(
        use_tc_tiling_on_sc=True, needs_layout_passes=False
    ),
)
def _emb_lookup(table_hbm, idx_hbm, out_hbm, table_v, idx_v, bufs, osems):
    wid = lax.axis_index("s") * NC + lax.axis_index("c")
    base = wid * BPW
    pltpu.sync_copy(table_hbm, table_v)
    pltpu.sync_copy(idx_hbm.at[pl.ds(base, BPW)], idx_v)

    lane_iota = lax.iota(jnp.int32, L)

    FPS = SLABR // STATIC_VARS
    OUTW = STATIC_VARS * DIM
    frow0 = wid * (BPW // STATIC_VARS)

    def drain_out(slot):
        for fr in range(FPS):
            pltpu.make_async_copy(
                bufs.at[pl.ds(slot * SLABR * DIM + fr * OUTW, OUTW)],
                out_hbm.at[frow0],
                osems.at[slot],
            ).wait()

    def body(s, _):
        slot = s % K

        @pl.when(s >= K)
        def _():
            drain_out(slot)

        srow = s * SLABR
        sbuf = slot * SLABR * DIM

        @plsc.parallel_loop(0, SLABR, unroll=UNROLL)
        def row(r):
            rsplat = plsc.load_gather(
                idx_v, [jnp.full((L,), srow + r, jnp.int32)]
            ).astype(jnp.int32)
            gbase = rsplat * DIM + lane_iota
            obase = sbuf + r * DIM
            for q in range(DIM // L):
                vals = plsc.load_gather(table_v, [gbase + q * L])
                bufs[pl.ds(obase + q * L, L)] = vals

        for fr in range(FPS):
            pltpu.async_copy(
                bufs.at[pl.ds(sbuf + fr * OUTW, OUTW)],
                out_hbm.at[frow0 + s * FPS + fr],
                osems.at[slot],
            )
        return 0

    lax.fori_loop(0, NSL, body, 0)
    for t in range(K):
        drain_out((NSL - K + t) % K)


def kernel(static_input, table):
    idx = static_input.astype(jnp.int32).reshape(B)
    return _emb_lookup(table.astype(jnp.float32).reshape(-1), idx)


# speedup vs baseline: 1.4221x; 1.0194x over previous
import functools

import jax
import jax.numpy as jnp
from jax import lax
from jax.experimental import pallas as pl
from jax.experimental.pallas import tpu as pltpu
from jax.experimental.pallas import tpu_sc as plsc

STATIC_VARS = 26
DIM = 64
BATCH = 4096
B = BATCH * STATIC_VARS
NC, NS = 2, 16
NW = NC * NS
BPW = B // NW
FPW = BPW // STATIC_VARS
L = 16
SLABR = 416
FPS = SLABR // STATIC_VARS
OUTW = STATIC_VARS * DIM
NSL = BPW // SLABR
K = 2
UNROLL = 8

_MESH = plsc.VectorSubcoreMesh(
    core_axis_name="c", subcore_axis_name="s", num_cores=NC, num_subcores=NS
)


@functools.partial(
    pl.kernel,
    out_type=jax.ShapeDtypeStruct((BATCH, OUTW), jnp.float32),
    mesh=_MESH,
    scratch_types=[
        pltpu.VMEM((STATIC_VARS, DIM), jnp.float32),
        pltpu.VMEM((FPW, STATIC_VARS), jnp.int32),
        pltpu.VMEM((STATIC_VARS * DIM,), jnp.float32),
        pltpu.VMEM((BPW,), jnp.int32),
        pltpu.VMEM((K * SLABR * DIM,), jnp.float32),
        pltpu.SemaphoreType.DMA((K,)),
    ],
    compiler_params=pltpu.CompilerParams(
        use_tc_tiling_on_sc=True, needs_layout_passes=False
    ),
)
def _emb_lookup(table_hbm, idx_hbm, out_hbm, table2d_v, idx2d_v, table_v,
                idx_v, bufs, osems):
    wid = lax.axis_index("s") * NC + lax.axis_index("c")
    frow0 = wid * FPW
    pltpu.sync_copy(table_hbm, table2d_v)
    pltpu.sync_copy(idx_hbm.at[pl.ds(frow0, FPW)], idx2d_v)

    lane_iota = lax.iota(jnp.int32, L)

    for t in range(STATIC_VARS):
        tsplat = jnp.full((L,), t, jnp.int32)
        for q in range(DIM // L):
            table_v[pl.ds(t * DIM + q * L, L)] = plsc.load_gather(
                table2d_v, [tsplat, lane_iota + q * L]
            )

    @plsc.parallel_loop(0, FPW, unroll=4)
    def depad(fl):
        flsplat = jnp.full((L,), fl, jnp.int32)
        idx_v[pl.ds(fl * STATIC_VARS, L)] = plsc.load_gather(
            idx2d_v, [flsplat, lane_iota]
        )
        idx_v[pl.ds(fl * STATIC_VARS + STATIC_VARS - L, L)] = plsc.load_gather(
            idx2d_v, [flsplat, lane_iota + (STATIC_VARS - L)]
        )

    def drain_out(slot):
        for fr in range(FPS):
            pltpu.make_async_copy(
                bufs.at[pl.ds(slot * SLABR * DIM + fr * OUTW, OUTW)],
                out_hbm.at[frow0],
                osems.at[slot],
            ).wait()

    def body(s, _):
        slot = s % K

        @pl.when(s >= K)
        def _():
            drain_out(slot)

        srow = s * SLABR
        sbuf = slot * SLABR * DIM

        @plsc.parallel_loop(0, SLABR, unroll=UNROLL)
        def row(r):
            rsplat = plsc.load_gather(
                idx_v, [jnp.full((L,), srow + r, jnp.int32)]
            )
            gbase = rsplat * DIM + lane_iota
            obase = sbuf + r * DIM
            for q in range(DIM // L):
                vals = plsc.load_gather(table_v, [gbase + q * L])
                bufs[pl.ds(obase + q * L, L)] = vals

        for fr in range(FPS):
            pltpu.async_copy(
                bufs.at[pl.ds(sbuf + fr * OUTW, OUTW)],
                out_hbm.at[frow0 + s * FPS + fr],
                osems.at[slot],
            )
        return 0

    lax.fori_loop(0, NSL, body, 0)
    for t in range(K):
        drain_out((NSL - K + t) % K)


def kernel(static_input, table):
    return _emb_lookup(table.astype(jnp.float32), static_input.astype(jnp.int32))
